# baseline (speedup 1.0000x reference)
import functools

import jax
import jax.numpy as jnp
from jax import lax
from jax.experimental import pallas as pl
from jax.experimental.pallas import tpu as pltpu
from jax.experimental.pallas import tpu_sc as plsc

B = 1024
L = 50
RDIM = 64
TDIM = 64
D = RDIM + TDIM
H = 256
OUT = 128

NCHUNK = 5
LC = L // NCHUNK
CB = LC * B

_NC = 2
_NS = 16
_NW = _NC * _NS
_BPW = CB // _NW
_CH = 80
_NCH = _BPW // _CH


def _sc_gather(idx3d, ctab):
    mesh = plsc.VectorSubcoreMesh(core_axis_name="c", subcore_axis_name="s")

    @functools.partial(
        pl.kernel,
        out_type=jax.ShapeDtypeStruct((CB, D), jnp.float32),
        mesh=mesh,
        scratch_types=(
            pltpu.VMEM((_NCH, _CH), jnp.int32),
            pltpu.VMEM((_BPW, D), jnp.float32),
            pltpu.SemaphoreType.DMA,
        ),
    )
    def k(idx_hbm, tab_hbm, out_hbm, idx_v, rows_v, sem):
        wid = lax.axis_index("s") * _NC + lax.axis_index("c")
        base = wid * _BPW
        pltpu.sync_copy(idx_hbm.at[wid], idx_v)
        descs = [
            pltpu.async_copy(
                tab_hbm.at[idx_v.at[j]],
                rows_v.at[pl.ds(j * _CH, _CH)],
                sem,
            )
            for j in range(_NCH)
        ]
        for d in descs:
            d.wait()
        pltpu.sync_copy(rows_v, out_hbm.at[pl.ds(base, _BPW)])

    return k(idx3d, ctab)


def _make_lstm_body(last):
    def body(x_ref, tv_ref, wih_ref, whh_ref, b_ref, fcw_ref, fcb_ref,
             h0_ref, c0_ref, ho_ref, co_ref, out_ref, h_scr, c_scr):
        t = pl.program_id(0)

        @pl.when(t == 0)
        def _():
            h_scr[...] = h0_ref[...]
            c_scr[...] = c0_ref[...]

        x = x_ref[0]
        tm = tv_ref[0]
        xs = jnp.concatenate(
            [x[:, :RDIM], x[:, RDIM:] * tm], axis=1
        ).astype(jnp.bfloat16)
        h = h_scr[...]
        gates = (
            jnp.dot(xs, wih_ref[...], preferred_element_type=jnp.float32)
            + jnp.dot(h.astype(jnp.bfloat16), whh_ref[...],
                      preferred_element_type=jnp.float32)
            + b_ref[...]
        )
        th = jnp.tanh(gates)
        i = th[:, :H] * 0.5 + 0.5
        f = th[:, H:2 * H] * 0.5 + 0.5
        g = th[:, 2 * H:3 * H]
        o = th[:, 3 * H:] * 0.5 + 0.5
        c = f * c_scr[...] + i * g
        hn = o * jnp.tanh(c)
        c_scr[...] = c
        h_scr[...] = hn

        @pl.when(t == LC - 1)
        def _():
            ho_ref[...] = hn
            co_ref[...] = c
            if last:
                out_ref[...] = jnp.tanh(
                    jnp.dot(hn.astype(jnp.bfloat16), fcw_ref[...],
                            preferred_element_type=jnp.float32)
                    + fcb_ref[...]
                ) * 0.5 + 0.5

    return body


def _lstm_tc(x, tv, wih_t, whh_t, bias, fcw_t, fcb, h0, c0, last):
    full = lambda t: (0, 0)
    return pl.pallas_call(
        _make_lstm_body(last),
        grid=(LC,),
        in_specs=[
            pl.BlockSpec((1, B, D), lambda t: (t, 0, 0)),
            pl.BlockSpec((1, B, TDIM), lambda t: (t, 0, 0)),
            pl.BlockSpec((D, 4 * H), full),
            pl.BlockSpec((H, 4 * H), full),
            pl.BlockSpec((1, 4 * H), full),
            pl.BlockSpec((H, OUT), full),
            pl.BlockSpec((1, OUT), full),
            pl.BlockSpec((B, H), full),
            pl.BlockSpec((B, H), full),
        ],
        out_specs=[
            pl.BlockSpec((B, H), full),
            pl.BlockSpec((B, H), full),
            pl.BlockSpec((B, OUT), full),
        ],
        out_shape=[
            jax.ShapeDtypeStruct((B, H), jnp.float32),
            jax.ShapeDtypeStruct((B, H), jnp.float32),
            jax.ShapeDtypeStruct((B, OUT), jnp.float32),
        ],
        scratch_shapes=[
            pltpu.VMEM((B, H), jnp.float32),
            pltpu.VMEM((B, H), jnp.float32),
        ],
    )(x, tv, wih_t, whh_t, bias, fcw_t, fcb, h0, c0)


def kernel(region_sequences, time_sequences, region_table, time_table,
           W_ih, W_hh, b_ih, b_hh, fc_W, fc_b):
    idx = jnp.transpose(region_sequences).reshape(NCHUNK, _NW, _NCH, _CH)
    ctab = jnp.concatenate([region_table, time_table], axis=1)
    tv = jnp.broadcast_to(
        jnp.transpose(time_sequences).reshape(L, B, 1), (L, B, TDIM)
    ).reshape(NCHUNK, LC, B, TDIM)
    colscale = jnp.concatenate(
        [jnp.full((2 * H,), 0.5, jnp.float32),
         jnp.ones((H,), jnp.float32),
         jnp.full((H,), 0.5, jnp.float32)]
    )
    wih_t = (W_ih.T * colscale[None, :]).astype(jnp.bfloat16)
    whh_t = (W_hh.T * colscale[None, :]).astype(jnp.bfloat16)
    bias = ((b_ih + b_hh) * colscale).reshape(1, 4 * H)
    fcw_t = (fc_W.T * 0.5).astype(jnp.bfloat16)
    fcb = (fc_b * 0.5).reshape(1, OUT)

    xs = [_sc_gather(idx[c], ctab).reshape(LC, B, D) for c in range(NCHUNK)]
    h = jnp.zeros((B, H), jnp.float32)
    c = jnp.zeros((B, H), jnp.float32)
    out = None
    for ci in range(NCHUNK):
        h, c, out = _lstm_tc(xs[ci], tv[ci], wih_t, whh_t, bias, fcw_t, fcb,
                             h, c, last=(ci == NCHUNK - 1))
    return out
